# baseline (speedup 1.0000x reference)
import functools

import jax
import jax.numpy as jnp
from jax import lax
from jax.experimental import pallas as pl
from jax.experimental.pallas import tpu as pltpu
from jax.experimental.pallas import tpu_sc as plsc

_N_SLOW = 8
_N_FAST = 32
_BC = 12
_SUB = 8
_SUBW = (224 * 224) // _SUB
_NC = 2
_NS = 16
_NW = _NC * _NS
_SLOW_PER_W = (_BC * _N_SLOW) // _NW
_FAST_PER_W = (_BC * _N_FAST) // _NW
_ROWS_PER_W = _SLOW_PER_W + _FAST_PER_W


def _sc_body(frames_hbm, table_hbm, slow_hbm, fast_hbm,
             idx_v, buf0, buf1, sem_in, sem_out):
    cid = lax.axis_index("c")
    sid = lax.axis_index("s")
    wid = sid * _NC + cid

    pltpu.sync_copy(table_hbm.at[wid], idx_v)

    bufs = (buf0, buf1)
    pending = [None, None]
    for j in range(_ROWS_PER_W):
        if j < _SLOW_PER_W:
            dst = slow_hbm.at[pl.ds((wid * _SLOW_PER_W + j) * _SUB, _SUB)]
        else:
            dst = fast_hbm.at[
                pl.ds((wid * _FAST_PER_W + (j - _SLOW_PER_W)) * _SUB, _SUB)]
        b = bufs[j % 2]
        if pending[j % 2] is not None:
            pending[j % 2].wait()
        idx = idx_v.at[pl.ds(j * _SUB, _SUB)]
        rd = pltpu.make_async_copy(frames_hbm.at[idx], b, sem_in)
        rd.start()
        rd.wait()
        wr = pltpu.make_async_copy(b, dst, sem_out)
        wr.start()
        pending[j % 2] = wr
    pending[0].wait()
    pending[1].wait()


def kernel(frames):
    B, C, T, H, W = frames.shape
    frames_sub = frames.reshape(B * C * T * _SUB, _SUBW)

    idx_slow = jnp.linspace(0.0, float(T - 1), _N_SLOW).astype(jnp.int32)
    idx_fast = jnp.linspace(0.0, float(T - 1), _N_FAST).astype(jnp.int32)
    base = jnp.arange(_BC, dtype=jnp.int32) * T
    plane_slow = (base[:, None] + idx_slow[None, :]).reshape(_NW, _SLOW_PER_W)
    plane_fast = (base[:, None] + idx_fast[None, :]).reshape(_NW, _FAST_PER_W)
    planes = jnp.concatenate([plane_slow, plane_fast], axis=1)
    sub = jnp.arange(_SUB, dtype=jnp.int32)
    table = planes[:, :, None] * _SUB + sub[None, None, :]
    table = jnp.concatenate(
        [table.reshape(_NW, _ROWS_PER_W * _SUB),
         jnp.zeros((_NW, _SUB), jnp.int32)], axis=1)

    mesh = plsc.VectorSubcoreMesh(core_axis_name="c", subcore_axis_name="s",
                                  num_cores=_NC, num_subcores=_NS)
    run = functools.partial(
        pl.kernel,
        out_type=(
            jax.ShapeDtypeStruct((_BC * _N_SLOW * _SUB, _SUBW), jnp.float32),
            jax.ShapeDtypeStruct((_BC * _N_FAST * _SUB, _SUBW), jnp.float32),
        ),
        mesh=mesh,
        scratch_types=[
            pltpu.VMEM((_NW * 4,), jnp.int32),
            pltpu.VMEM((_SUB, _SUBW), jnp.float32),
            pltpu.VMEM((_SUB, _SUBW), jnp.float32),
            pltpu.SemaphoreType.DMA,
            pltpu.SemaphoreType.DMA,
        ],
    )(_sc_body)
    slow2d, fast2d = run(frames_sub, table)
    slow = slow2d.reshape(B, C, _N_SLOW, H, W)
    fast = fast2d.reshape(B, C, _N_FAST, H, W)
    return (slow, fast)


# speedup vs baseline: 4.1496x; 4.1496x over previous
import functools

import jax
import jax.numpy as jnp
from jax import lax
from jax.experimental import pallas as pl
from jax.experimental.pallas import tpu as pltpu
from jax.experimental.pallas import tpu_sc as plsc

_N_SLOW = 8
_N_FAST = 32
_BC = 12
_NC = 2
_NS = 16
_NW = _NC * _NS
_SLOW_PER_W = (_BC * _N_SLOW) // _NW
_FAST_PER_W = (_BC * _N_FAST) // _NW
_ROWS_PER_W = _SLOW_PER_W + _FAST_PER_W


def _sc_body(frames_hbm, slow_hbm, fast_hbm, buf0, buf1, sem_in, sem_out):
    cid = lax.axis_index("c")
    sid = lax.axis_index("s")
    wid = sid * _NC + cid

    bufs = (buf0, buf1)
    pending = [None, None]
    for j in range(_ROWS_PER_W):
        if j < _SLOW_PER_W:
            s = wid * _SLOW_PER_W + j
            bc = s // _N_SLOW
            t = s % _N_SLOW
            src = frames_hbm.at[bc, t * 9]
            dst = slow_hbm.at[bc, t]
        else:
            s = wid * _FAST_PER_W + (j - _SLOW_PER_W)
            bc = s // _N_FAST
            t = s % _N_FAST
            src = frames_hbm.at[bc, 2 * t + jnp.where(t == _N_FAST - 1, 1, 0)]
            dst = fast_hbm.at[bc, t]
        b = bufs[j % 2]
        if pending[j % 2] is not None:
            pending[j % 2].wait()
        rd = pltpu.make_async_copy(src, b, sem_in)
        rd.start()
        rd.wait()
        wr = pltpu.make_async_copy(b, dst, sem_out)
        wr.start()
        pending[j % 2] = wr
    pending[0].wait()
    pending[1].wait()


def kernel(frames):
    B, C, T, H, W = frames.shape
    frames3d = frames.reshape(B * C, T, H, W)

    mesh = plsc.VectorSubcoreMesh(core_axis_name="c", subcore_axis_name="s",
                                  num_cores=_NC, num_subcores=_NS)
    run = functools.partial(
        pl.kernel,
        out_type=(
            jax.ShapeDtypeStruct((_BC, _N_SLOW, H, W), jnp.float32),
            jax.ShapeDtypeStruct((_BC, _N_FAST, H, W), jnp.float32),
        ),
        mesh=mesh,
        scratch_types=[
            pltpu.VMEM((H, W), jnp.float32),
            pltpu.VMEM((H, W), jnp.float32),
            pltpu.SemaphoreType.DMA,
            pltpu.SemaphoreType.DMA,
        ],
        compiler_params=pltpu.CompilerParams(use_tc_tiling_on_sc=True),
    )(_sc_body)
    slow4d, fast4d = run(frames3d)
    slow = slow4d.reshape(B, C, _N_SLOW, H, W)
    fast = fast4d.reshape(B, C, _N_FAST, H, W)
    return (slow, fast)
